# Initial kernel scaffold; baseline (speedup 1.0000x reference)
import jax
import jax.numpy as jnp
from jax.experimental import pallas as pl


def kernel(x, edge_index, edge_attr, W):
    raise NotImplementedError("write your pallas kernel here")


# speedup vs baseline: 3.0799x; 3.0799x over previous
import functools

import jax
import jax.numpy as jnp
from jax import lax
from jax.experimental import pallas as pl
from jax.experimental.pallas import tpu as pltpu
from jax.experimental.pallas import tpu_sc as plsc


def _tables_body(x_ref, wab_ref, out_ref):
    x = x_ref[...]
    dn = (((1,), (1,)), ((), ()))
    out_ref[0] = lax.dot_general(x, wab_ref[0], dn,
                                 preferred_element_type=jnp.float32)
    out_ref[1] = lax.dot_general(x, wab_ref[1], dn,
                                 preferred_element_type=jnp.float32)


def _node_tables(x, w1, w2):
    n, d = x.shape
    wab = jnp.stack([w1, w2])
    return pl.pallas_call(
        _tables_body,
        out_shape=jax.ShapeDtypeStruct((2, n, d), jnp.float32),
    )(x, wab)


def _edge_mm_body(ea_ref, w3_ref, out_ref):
    dn = (((1,), (1,)), ((), ()))
    out_ref[...] = lax.dot_general(ea_ref[...], w3_ref[...], dn,
                                   preferred_element_type=jnp.float32)


def _edge_term(edge_attr, w3, block_e):
    e, d = edge_attr.shape
    grid = (e // block_e,)
    return pl.pallas_call(
        _edge_mm_body,
        grid=grid,
        in_specs=[
            pl.BlockSpec((block_e, d), lambda i: (i, 0)),
            pl.BlockSpec((d, d), lambda i: (0, 0)),
        ],
        out_specs=pl.BlockSpec((block_e, d), lambda i: (i, 0)),
        out_shape=jax.ShapeDtypeStruct((e, d), jnp.float32),
    )(edge_attr, w3)


def _make_sc_gather_add(e, n2, d, chunk, num_chunks_per_worker, nc, ns):
    epw = e // (nc * ns)
    mesh = plsc.VectorSubcoreMesh(core_axis_name="c", subcore_axis_name="s")

    @functools.partial(
        pl.kernel,
        out_type=jax.ShapeDtypeStruct((e, d), jnp.float32),
        mesh=mesh,
        scratch_types=[
            pltpu.VMEM((epw,), jnp.int32),
            pltpu.VMEM((epw,), jnp.int32),
            pltpu.VMEM((chunk, d), jnp.float32),
            pltpu.VMEM((chunk, d), jnp.float32),
            pltpu.VMEM((chunk, d), jnp.float32),
            pltpu.SemaphoreType.DMA,
            pltpu.SemaphoreType.DMA,
            pltpu.SemaphoreType.DMA,
        ],
    )
    def sc_kernel(table_hbm, src_hbm, dst_hbm, ea3_hbm, out_hbm,
                  idx_s, idx_d, rows_s, rows_d, rows_e, sem_s, sem_d, sem_e):
        wid = lax.axis_index("s") * nc + lax.axis_index("c")
        base_w = wid * epw
        pltpu.sync_copy(src_hbm.at[pl.ds(base_w, epw)], idx_s)
        pltpu.sync_copy(dst_hbm.at[pl.ds(base_w, epw)], idx_d)

        def chunk_body(c, carry):
            off = c * chunk
            base = base_w + off
            cp_s = pltpu.async_copy(
                table_hbm.at[idx_s.at[pl.ds(off, chunk)]], rows_s, sem_s)
            cp_d = pltpu.async_copy(
                table_hbm.at[idx_d.at[pl.ds(off, chunk)]], rows_d, sem_d)
            cp_e = pltpu.async_copy(
                ea3_hbm.at[pl.ds(base, chunk)], rows_e, sem_e)
            cp_s.wait()
            cp_d.wait()
            cp_e.wait()

            def row_body(r, carry2):
                for j in range(d // 16):
                    sl = pl.ds(j * 16, 16)
                    rows_s[r, sl] = rows_s[r, sl] + rows_d[r, sl] + rows_e[r, sl]
                return carry2

            lax.fori_loop(0, chunk, row_body, 0, unroll=False)
            pltpu.sync_copy(rows_s, out_hbm.at[pl.ds(base, chunk)])
            return carry

        lax.fori_loop(0, num_chunks_per_worker, chunk_body, 0, unroll=False)

    return sc_kernel


def kernel(x, edge_index, edge_attr, W):
    n, d = x.shape
    e = edge_attr.shape[0]
    w1 = W[:, :d]
    w2 = W[:, d:2 * d]
    w3 = W[:, 2 * d:]

    tables = _node_tables(x, w1, w2).reshape(2 * n, d)
    ea3 = _edge_term(edge_attr, w3, block_e=4000)

    src = edge_index[0]
    dstn = edge_index[1] + n

    nc, ns = 2, 16
    chunk = 80
    epw = e // (nc * ns)
    num_chunks = epw // chunk
    sc = _make_sc_gather_add(e, 2 * n, d, chunk, num_chunks, nc, ns)
    return sc(tables, src, dstn, ea3)


# speedup vs baseline: 4.2050x; 1.3653x over previous
import functools

import jax
import jax.numpy as jnp
from jax import lax
from jax.experimental import pallas as pl
from jax.experimental.pallas import tpu as pltpu
from jax.experimental.pallas import tpu_sc as plsc


def _tables_body(x_ref, wab_ref, out_ref):
    x = x_ref[...]
    dn = (((1,), (1,)), ((), ()))
    out_ref[0] = lax.dot_general(x, wab_ref[0], dn,
                                 preferred_element_type=jnp.float32)
    out_ref[1] = lax.dot_general(x, wab_ref[1], dn,
                                 preferred_element_type=jnp.float32)


def _node_tables(x, w1, w2):
    n, d = x.shape
    wab = jnp.stack([w1, w2])
    return pl.pallas_call(
        _tables_body,
        out_shape=jax.ShapeDtypeStruct((2, n, d), jnp.float32),
    )(x, wab)


def _edge_mm_body(ea_ref, w3_ref, out_ref):
    dn = (((1,), (1,)), ((), ()))
    out_ref[...] = lax.dot_general(ea_ref[...], w3_ref[...], dn,
                                   preferred_element_type=jnp.float32)


def _edge_term(edge_attr, w3, block_e):
    e, d = edge_attr.shape
    grid = (e // block_e,)
    return pl.pallas_call(
        _edge_mm_body,
        grid=grid,
        in_specs=[
            pl.BlockSpec((block_e, d), lambda i: (i, 0)),
            pl.BlockSpec((d, d), lambda i: (0, 0)),
        ],
        out_specs=pl.BlockSpec((block_e, d), lambda i: (i, 0)),
        out_shape=jax.ShapeDtypeStruct((e, d), jnp.float32),
    )(edge_attr, w3)


def _make_sc_gather_add(e, n2, d, chunk, nc, ns):
    epw = e // (nc * ns)
    nchunks = epw // chunk
    mesh = plsc.VectorSubcoreMesh(core_axis_name="c", subcore_axis_name="s")
    buf = lambda: pltpu.VMEM((chunk, d), jnp.float32)

    @functools.partial(
        pl.kernel,
        out_type=jax.ShapeDtypeStruct((e, d), jnp.float32),
        mesh=mesh,
        scratch_types=[
            pltpu.VMEM((epw,), jnp.int32),
            pltpu.VMEM((epw,), jnp.int32),
            buf(), buf(), buf(), buf(),
            buf(), buf(), buf(), buf(),
            pltpu.SemaphoreType.DMA,
            pltpu.SemaphoreType.DMA,
            pltpu.SemaphoreType.DMA,
            pltpu.SemaphoreType.DMA,
        ],
    )
    def sc_kernel(table_hbm, src_hbm, dst_hbm, ea3_hbm, out_hbm,
                  idx_s, idx_d,
                  s0, d0, e0, o0, s1, d1, e1, o1,
                  sem0, sem1, semw0, semw1):
        wid = lax.axis_index("s") * nc + lax.axis_index("c")
        base_w = wid * epw
        pltpu.sync_copy(src_hbm.at[pl.ds(base_w, epw)], idx_s)
        pltpu.sync_copy(dst_hbm.at[pl.ds(base_w, epw)], idx_d)

        sets = ((s0, d0, e0, o0, sem0, semw0), (s1, d1, e1, o1, sem1, semw1))

        def in_copies(c, st):
            bs, bd, be, _, sem, _ = st
            off = c * chunk
            return (
                pltpu.make_async_copy(
                    table_hbm.at[idx_s.at[pl.ds(off, chunk)]], bs, sem),
                pltpu.make_async_copy(
                    table_hbm.at[idx_d.at[pl.ds(off, chunk)]], bd, sem),
                pltpu.make_async_copy(
                    ea3_hbm.at[pl.ds(base_w + off, chunk)], be, sem),
            )

        def compute(st):
            bs, bd, be, bo, _, _ = st

            def row_body(r, carry):
                for j in range(d // 16):
                    sl = pl.ds(j * 16, 16)
                    bo[r, sl] = bs[r, sl] + bd[r, sl] + be[r, sl]
                return carry

            lax.fori_loop(0, chunk, row_body, 0, unroll=False)

        def store(c, st):
            _, _, _, bo, _, semw = st
            return pltpu.make_async_copy(
                bo, out_hbm.at[pl.ds(base_w + c * chunk, chunk)], semw)

        for cp in in_copies(0, sets[0]):
            cp.start()

        def pair_body(i, carry):
            ca = 2 * i
            cb = 2 * i + 1
            for cp in in_copies(cb, sets[1]):
                cp.start()
            for cp in in_copies(ca, sets[0]):
                cp.wait()

            @pl.when(i > 0)
            def _():
                store(ca, sets[0]).wait()

            compute(sets[0])
            store(ca, sets[0]).start()

            @pl.when(cb + 1 < nchunks)
            def _():
                for cp in in_copies(cb + 1, sets[0]):
                    cp.start()

            for cp in in_copies(cb, sets[1]):
                cp.wait()

            @pl.when(i > 0)
            def _():
                store(cb, sets[1]).wait()

            compute(sets[1])
            store(cb, sets[1]).start()
            return carry

        lax.fori_loop(0, nchunks // 2, pair_body, 0, unroll=False)
        store(nchunks - 2, sets[0]).wait()
        store(nchunks - 1, sets[1]).wait()

    return sc_kernel


def kernel(x, edge_index, edge_attr, W):
    n, d = x.shape
    e = edge_attr.shape[0]
    w1 = W[:, :d]
    w2 = W[:, d:2 * d]
    w3 = W[:, 2 * d:]

    tables = _node_tables(x, w1, w2).reshape(2 * n, d)
    ea3 = _edge_term(edge_attr, w3, block_e=4000)

    src = edge_index[0]
    dstn = edge_index[1] + n

    nc, ns = 2, 16
    chunk = 40
    sc = _make_sc_gather_add(e, 2 * n, d, chunk, nc, ns)
    return sc(tables, src, dstn, ea3)
